# baseline (speedup 1.0000x reference)
import functools

import jax
import jax.numpy as jnp
from jax.experimental import pallas as pl

_N = 262144
_B = 64
_M = 64
_TN = 8192
_LN_EPS = -36.841361487904734


def _softplus(x):
    return jnp.logaddexp(x, 0.0)


def _block_logits(key_ref, strength_ref, mem_ref):
    key = key_ref[...]
    beta = _softplus(strength_ref[...])
    key_n = key / (jnp.sqrt(jnp.sum(key * key, axis=1, keepdims=True)) + 1e-16)
    q = beta * key_n
    mem = mem_ref[...]
    dots = jax.lax.dot_general(
        q, mem, (((1,), (1,)), ((), ())),
        preferred_element_type=jnp.float32,
        precision=jax.lax.Precision.HIGHEST)
    ones = jnp.ones((1, _M), jnp.float32)
    nsq = jax.lax.dot_general(
        ones, mem * mem, (((1,), (1,)), ((), ())),
        preferred_element_type=jnp.float32,
        precision=jax.lax.Precision.HIGHEST)
    rnorm = 1.0 / (jnp.sqrt(nsq) + 1e-16)
    return dots * rnorm - beta


def _stats_kernel(key_ref, strength_ref, sharpen_ref, mem_ref, s1_ref, s2_ref):
    j = pl.program_id(0)
    t = _block_logits(key_ref, strength_ref, mem_ref)
    gamma = 1.0 + _softplus(sharpen_ref[...])
    p1 = jnp.sum(jnp.exp(t), axis=1, keepdims=True)
    p2 = jnp.sum(jnp.exp(gamma * t), axis=1, keepdims=True)

    @pl.when(j == 0)
    def _():
        s1_ref[...] = p1
        s2_ref[...] = p2

    @pl.when(j > 0)
    def _():
        s1_ref[...] += p1
        s2_ref[...] += p2


def _out_kernel(key_ref, strength_ref, sharpen_ref, s1_ref, s2_ref, mem_ref,
                out_ref):
    t = _block_logits(key_ref, strength_ref, mem_ref)
    gamma = 1.0 + _softplus(sharpen_ref[...])
    eps_term = jnp.exp(gamma * jnp.log(s1_ref[...]) + _LN_EPS)
    inv_d = 1.0 / (s2_ref[...] + eps_term)
    out_ref[...] = jnp.exp(gamma * t) * inv_d


@jax.jit
def kernel(key, strength, sharpen, memory):
    grid = (_N // _TN,)
    small = [
        pl.BlockSpec((_B, _M), lambda j: (0, 0)),
        pl.BlockSpec((_B, 1), lambda j: (0, 0)),
        pl.BlockSpec((_B, 1), lambda j: (0, 0)),
    ]
    mem_spec = pl.BlockSpec((_TN, _M), lambda j: (j, 0))
    stat_spec = pl.BlockSpec((_B, 1), lambda j: (0, 0))

    s1, s2 = pl.pallas_call(
        _stats_kernel,
        grid=grid,
        in_specs=small + [mem_spec],
        out_specs=[stat_spec, stat_spec],
        out_shape=[jax.ShapeDtypeStruct((_B, 1), jnp.float32)] * 2,
    )(key, strength, sharpen, memory)

    out = pl.pallas_call(
        _out_kernel,
        grid=grid,
        in_specs=small + [stat_spec, stat_spec, mem_spec],
        out_specs=pl.BlockSpec((_B, _TN), lambda j: (0, j)),
        out_shape=jax.ShapeDtypeStruct((_B, _N), jnp.float32),
    )(key, strength, sharpen, s1, s2, memory)
    return out


# speedup vs baseline: 2.1161x; 2.1161x over previous
import jax
import jax.numpy as jnp
from jax.experimental import pallas as pl

_N = 262144
_B = 64
_M = 64
_TN = 8192
_LN_EPS = -36.841361487904734


def _softplus(x):
    return jnp.logaddexp(x, 0.0)


def _prep_kernel(key_ref, strength_ref, sharpen_ref, q_ref, beta_ref,
                 gamma_ref):
    key = key_ref[...]
    beta = _softplus(strength_ref[...])
    key_n = key / (jnp.sqrt(jnp.sum(key * key, axis=1, keepdims=True)) + 1e-16)
    q_ref[...] = beta * key_n
    beta_ref[...] = beta
    gamma_ref[...] = 1.0 + _softplus(sharpen_ref[...])


def _split_bf16(x):
    hi = x.astype(jnp.bfloat16)
    lo = (x - hi.astype(jnp.float32)).astype(jnp.bfloat16)
    return hi, lo


def _dots(q, mem):
    q_hi, q_lo = _split_bf16(q)
    m_hi, m_lo = _split_bf16(mem)
    dn = (((1,), (1,)), ((), ()))
    kw = dict(preferred_element_type=jnp.float32)
    return (jax.lax.dot_general(q_hi, m_hi, dn, **kw)
            + jax.lax.dot_general(q_hi, m_lo, dn, **kw)
            + jax.lax.dot_general(q_lo, m_hi, dn, **kw))


def _stats_kernel(q_ref, beta_ref, gamma_ref, mem_ref, s1_ref, s2_ref,
                  rnorm_ref):
    j = pl.program_id(0)
    mem = mem_ref[...]
    dots = _dots(q_ref[...], mem)
    ones = jnp.ones((1, _M), jnp.float32)
    nsq = jax.lax.dot_general(
        ones, mem * mem, (((1,), (1,)), ((), ())),
        preferred_element_type=jnp.float32)
    rnorm = 1.0 / (jnp.sqrt(nsq) + 1e-16)
    rnorm_ref[...] = rnorm
    t = dots * rnorm - beta_ref[...]
    gamma = gamma_ref[...]
    p1 = jnp.sum(jnp.exp(t), axis=1, keepdims=True)
    p2 = jnp.sum(jnp.exp(gamma * t), axis=1, keepdims=True)

    @pl.when(j == 0)
    def _():
        s1_ref[...] = p1
        s2_ref[...] = p2

    @pl.when(j > 0)
    def _():
        s1_ref[...] += p1
        s2_ref[...] += p2


def _out_kernel(q_ref, beta_ref, gamma_ref, s1_ref, s2_ref, rnorm_ref,
                mem_ref, out_ref):
    dots = _dots(q_ref[...], mem_ref[...])
    t = dots * rnorm_ref[...] - beta_ref[...]
    gamma = gamma_ref[...]
    eps_term = jnp.exp(gamma * jnp.log(s1_ref[...]) + _LN_EPS)
    inv_d = 1.0 / (s2_ref[...] + eps_term)
    out_ref[...] = jnp.exp(gamma * t) * inv_d


@jax.jit
def kernel(key, strength, sharpen, memory):
    col = jax.ShapeDtypeStruct((_B, 1), jnp.float32)
    q, beta, gamma = pl.pallas_call(
        _prep_kernel,
        out_shape=[jax.ShapeDtypeStruct((_B, _M), jnp.float32), col, col],
    )(key, strength, sharpen)

    grid = (_N // _TN,)
    small = [
        pl.BlockSpec((_B, _M), lambda j: (0, 0)),
        pl.BlockSpec((_B, 1), lambda j: (0, 0)),
        pl.BlockSpec((_B, 1), lambda j: (0, 0)),
    ]
    mem_spec = pl.BlockSpec((_TN, _M), lambda j: (j, 0))
    stat_spec = pl.BlockSpec((_B, 1), lambda j: (0, 0))
    rnorm_spec = pl.BlockSpec((1, _TN), lambda j: (0, j))

    s1, s2, rnorm = pl.pallas_call(
        _stats_kernel,
        grid=grid,
        in_specs=small + [mem_spec],
        out_specs=[stat_spec, stat_spec, rnorm_spec],
        out_shape=[col, col, jax.ShapeDtypeStruct((1, _N), jnp.float32)],
    )(q, beta, gamma, memory)

    out = pl.pallas_call(
        _out_kernel,
        grid=grid,
        in_specs=small + [stat_spec, stat_spec, rnorm_spec, mem_spec],
        out_specs=pl.BlockSpec((_B, _TN), lambda j: (0, j)),
        out_shape=jax.ShapeDtypeStruct((_B, _N), jnp.float32),
    )(q, beta, gamma, s1, s2, rnorm, memory)
    return out
